# Initial kernel scaffold; baseline (speedup 1.0000x reference)
import jax
import jax.numpy as jnp
from jax.experimental import pallas as pl


def kernel(hidden_states, ln_g, ln_b, Wr, br, Wu, bu, Wd, bd):
    raise NotImplementedError("write your pallas kernel here")


# speedup vs baseline: 3.0061x; 3.0061x over previous
import functools

import jax
import jax.numpy as jnp
from jax.experimental import pallas as pl
from jax.experimental.pallas import tpu as pltpu

E = 8
TOPK = 2
BLK = 512
BT = 512
IC = 1024


def _router_body(x_ref, g_ref, b_ref, wr_ref, br_ref, ew_ref, ei_ref):
    x = x_ref[...]
    mu = jnp.mean(x, axis=1, keepdims=True)
    var = jnp.mean((x - mu) ** 2, axis=1, keepdims=True)
    norm = (x - mu) * jax.lax.rsqrt(var + 1e-5) * g_ref[...] + b_ref[...]
    logits = jnp.dot(norm, wr_ref[...], preferred_element_type=jnp.float32)
    logits = logits + br_ref[...]
    iota = jax.lax.broadcasted_iota(jnp.int32, logits.shape, 1)
    m1 = jnp.max(logits, axis=1, keepdims=True)
    a1 = jnp.min(jnp.where(logits == m1, iota, E), axis=1, keepdims=True)
    masked = jnp.where(iota == a1, -jnp.inf, logits)
    m2 = jnp.max(masked, axis=1, keepdims=True)
    a2 = jnp.min(jnp.where(masked == m2, iota, E), axis=1, keepdims=True)
    t = jnp.exp(m2 - m1)
    w1 = 1.0 / (1.0 + t)
    ew_ref[...] = jnp.concatenate([w1, 1.0 - w1], axis=1)
    ei_ref[...] = jnp.concatenate([a1, a2], axis=1)


def _ffn_body(be_ref, na_ref, xs_ref, wu_ref, bu_ref, wd_ref, bd_ref,
              ws_ref, ys_ref):
    b = pl.program_id(0)

    @pl.when(b < na_ref[0])
    def _():
        inter = wu_ref.shape[2]
        xb = xs_ref[...].astype(jnp.bfloat16)
        acc = jnp.zeros((BLK, xs_ref.shape[1]), jnp.float32)
        for c in range(inter // IC):
            sl = pl.ds(c * IC, IC)
            h = jnp.dot(xb, wu_ref[0, :, sl],
                        preferred_element_type=jnp.float32)
            h = h + bu_ref[0, 0, sl]
            h = 0.5 * h * (1.0 + jax.lax.erf(h * (2.0 ** -0.5)))
            acc = acc + jnp.dot(h.astype(jnp.bfloat16), wd_ref[0, sl, :],
                                preferred_element_type=jnp.float32)
        y = (acc + bd_ref[0, 0, :]) * ws_ref[0, 0, :][:, None]
        ys_ref[...] = y


def kernel(hidden_states, ln_g, ln_b, Wr, br, Wu, bu, Wd, bd):
    Bsz, Sq, H = hidden_states.shape
    T = Bsz * Sq
    A = T * TOPK
    NB = A // BLK + E
    CAP = NB * BLK
    inter = Wu.shape[2]
    x = hidden_states.reshape(T, H)

    ew, ei = pl.pallas_call(
        _router_body,
        grid=(T // BT,),
        in_specs=[
            pl.BlockSpec((BT, H), lambda i: (i, 0)),
            pl.BlockSpec((H,), lambda i: (0,)),
            pl.BlockSpec((H,), lambda i: (0,)),
            pl.BlockSpec((H, E), lambda i: (0, 0)),
            pl.BlockSpec((E,), lambda i: (0,)),
        ],
        out_specs=[
            pl.BlockSpec((BT, TOPK), lambda i: (i, 0)),
            pl.BlockSpec((BT, TOPK), lambda i: (i, 0)),
        ],
        out_shape=[
            jax.ShapeDtypeStruct((T, TOPK), jnp.float32),
            jax.ShapeDtypeStruct((T, TOPK), jnp.int32),
        ],
    )(x, ln_g, ln_b, Wr, br)

    ef = ei.reshape(A)
    wf = ew.reshape(A)
    onehot = (ef[:, None] == jnp.arange(E, dtype=jnp.int32)[None, :]
              ).astype(jnp.int32)
    rank = jnp.take_along_axis(jnp.cumsum(onehot, axis=0) - onehot,
                               ef[:, None], axis=1)[:, 0]
    counts = jnp.sum(onehot, axis=0)
    padded = ((counts + BLK - 1) // BLK) * BLK
    cum = jnp.cumsum(padded)
    pos = (cum - padded)[ef] + rank
    tokens = jnp.arange(A, dtype=jnp.int32) // TOPK
    src = jnp.zeros((CAP,), jnp.int32).at[pos].set(tokens)
    ws = jnp.zeros((CAP,), jnp.float32).at[pos].set(wf)
    be = jnp.minimum(
        jnp.searchsorted(cum, jnp.arange(NB, dtype=jnp.int32) * BLK,
                         side="right"),
        E - 1).astype(jnp.int32)
    na = (cum[-1] // BLK).astype(jnp.int32).reshape(1)

    xs = jnp.take(x, src, axis=0)

    grid_spec = pltpu.PrefetchScalarGridSpec(
        num_scalar_prefetch=2,
        grid=(NB,),
        in_specs=[
            pl.BlockSpec((BLK, H), lambda b, be, na: (b, 0)),
            pl.BlockSpec((1, H, inter), lambda b, be, na: (be[b], 0, 0)),
            pl.BlockSpec((1, 1, inter), lambda b, be, na: (be[b], 0, 0)),
            pl.BlockSpec((1, inter, H), lambda b, be, na: (be[b], 0, 0)),
            pl.BlockSpec((1, 1, H), lambda b, be, na: (be[b], 0, 0)),
            pl.BlockSpec((1, 1, BLK), lambda b, be, na: (b, 0, 0)),
        ],
        out_specs=pl.BlockSpec((BLK, H), lambda b, be, na: (b, 0)),
    )
    ys = pl.pallas_call(
        _ffn_body,
        grid_spec=grid_spec,
        out_shape=jax.ShapeDtypeStruct((CAP, H), jnp.float32),
    )(be, na, xs, Wu.astype(jnp.bfloat16), bu.reshape(E, 1, inter),
      Wd.astype(jnp.bfloat16), bd.reshape(E, 1, H), ws.reshape(NB, 1, BLK))

    pos2 = pos.reshape(T, TOPK)
    out = jnp.take(ys, pos2[:, 0], axis=0) + jnp.take(ys, pos2[:, 1], axis=0)
    return out.reshape(Bsz, Sq, H)
